# baseline (speedup 1.0000x reference)
import functools

import jax
import jax.numpy as jnp
from jax import lax
from jax.experimental import pallas as pl
from jax.experimental.pallas import tpu as pltpu
from jax.experimental.pallas import tpu_sc as plsc

_B = 16384
_H = 50
_D = 32
_V = 1000000
_NC = 2
_NS = 16
_NW = _NC * _NS
_BPW = _B // _NW
_CB = 32
_NCHUNK = _BPW // _CB

_RB = 128
_SB = 256
_NBLK = _V // _SB
_TAIL = _V - _NBLK * _SB
_BLK_STATIC = 123
_BLK_LAST = _NBLK - _BLK_STATIC


def _transpose_body(tt_hbm, tail_hbm, out_hbm,
                    sbuf0, sbuf1, obuf0, obuf1,
                    isem0, isem1, osem0, osem1):
    wid = lax.axis_index("s") * _NC + lax.axis_index("c")
    base = jnp.minimum(wid * _BLK_STATIC, _BLK_LAST)
    sbufs = (sbuf0, sbuf1)
    obufs = (obuf0, obuf1)
    isems = (isem0, isem1)
    osems = (osem0, osem1)

    lanes = lax.iota(jnp.int32, 16)
    z = jnp.minimum(wid, 0)

    def in_copy(t, p):
        c0 = pl.multiple_of(t * _SB, _SB)
        return pltpu.make_async_copy(
            tt_hbm.at[:, pl.ds(c0, _SB)], sbufs[p], isems[p])

    def out_copy(t, p):
        s0 = pl.multiple_of(t * (_SB // 4), _SB // 4)
        return pltpu.make_async_copy(
            obufs[p], out_hbm.at[pl.ds(s0, _SB // 4)], osems[p])

    def transpose_block(p):
        sb = sbufs[p]
        ob = obufs[p]

        @plsc.parallel_loop(0, 16, 1, unroll=4)
        def _(d):
            m = jnp.bitwise_and(lanes + d, 15)
            mrow = jnp.right_shift(m, 2)
            mcol = jnp.bitwise_and(m, 3) * _D
            for cg in range(2):
                lc = lanes + cg * 16
                colf = mcol + lc
                srcs = [plsc.load_gather(sb, [lc, m + rg * 16])
                        for rg in range(_SB // 16)]
                for rg in range(_SB // 16):
                    plsc.store_scatter(ob, [mrow + rg * 4, colf], srcs[rg])

    def do_block(t, p, first):
        in_copy(t, p).wait()
        @pl.when(jnp.logical_not(first))
        def _():
            out_copy(t, p).wait()
        transpose_block(p)
        out_copy(t, p).start()
        nxt = t + 2
        @pl.when(nxt < base + _BLK_STATIC)
        def _():
            in_copy(nxt, p).start()

    in_copy(base, 0).start()
    in_copy(base + 1, 1).start()

    def pair_body(j, carry):
        t = base + 2 * j
        do_block(t, 0, j == 0)
        do_block(t + 1, 1, j == 0)
        return carry

    lax.fori_loop(0, _BLK_STATIC // 2, pair_body, 0)
    do_block(base + _BLK_STATIC - 1, 0, False)
    out_copy(base + _BLK_STATIC - 1, 0).wait()
    out_copy(base + _BLK_STATIC - 2, 1).wait()

    @pl.when(wid == _NW - 1)
    def _():
        pltpu.async_copy(tail_hbm, obuf0.at[pl.ds(0, 16)], isem0).wait()
        pltpu.async_copy(obuf0.at[pl.ds(0, 16)],
                         out_hbm.at[pl.ds(_NBLK * (_SB // 4), 16)],
                         isem0).wait()


def _bow_body(idx_hbm, table_hbm, bias_hbm, out_hbm,
              idx0, idx1, rows0, rows1, outv0, outv1, bias_v,
              gsem0, gsem1, osem0, osem1):
    wid = lax.axis_index("s") * _NC + lax.axis_index("c")
    base_row = wid * _BPW
    idxs = (idx0, idx1)
    rows = (rows0, rows1)
    outs = (outv0, outv1)
    gsems = (gsem0, gsem1)
    osems = (osem0, osem1)

    pltpu.sync_copy(bias_hbm, bias_v)
    bias0 = bias_v[pl.ds(0, 16)]
    bias1 = bias_v[pl.ds(16, 16)]

    def row0_of(c):
        return pl.multiple_of(base_row + c * _CB, _CB)

    def fire(c, p):
        pltpu.sync_copy(idx_hbm.at[pl.ds(row0_of(c), _CB)], idxs[p])
        for r in range(_CB):
            pltpu.make_async_copy(
                table_hbm.at[idxs[p].at[r]],
                rows[p].at[pl.ds(r * _H, _H)], gsems[p]).start()

    def drain(p):
        for r in range(_CB):
            pltpu.make_async_copy(
                table_hbm.at[idxs[p].at[r]],
                rows[p].at[pl.ds(r * _H, _H)], gsems[p]).wait()

    def out_copy(c, p):
        return pltpu.make_async_copy(
            outs[p], out_hbm.at[pl.ds(row0_of(c), _CB)], osems[p])

    def accumulate(p):
        rv = rows[p]
        ov = outs[p]

        @plsc.parallel_loop(0, _CB, 1, unroll=2)
        def _(b):
            r0 = b * _H
            a0 = bias0
            a1 = bias1
            b0 = rv[r0, pl.ds(0, 16)]
            b1 = rv[r0, pl.ds(16, 16)]
            for h in range(1, _H, 2):
                a0 = a0 + rv[r0 + h, pl.ds(0, 16)]
                a1 = a1 + rv[r0 + h, pl.ds(16, 16)]
                if h + 1 < _H:
                    b0 = b0 + rv[r0 + h + 1, pl.ds(0, 16)]
                    b1 = b1 + rv[r0 + h + 1, pl.ds(16, 16)]
            ov[b, pl.ds(0, 16)] = a0 + b0
            ov[b, pl.ds(16, 16)] = a1 + b1

    def step(c, p, first):
        drain(p)
        @pl.when(c + 1 < _NCHUNK)
        def _():
            fire(c + 1, 1 - p)
        @pl.when(jnp.logical_not(first))
        def _():
            out_copy(c, p).wait()
        accumulate(p)
        out_copy(c, p).start()

    fire(0, 0)

    def pair_body(j, carry):
        c = 2 * j
        step(c, 0, j == 0)
        step(c + 1, 1, j == 0)
        return carry

    lax.fori_loop(0, _NCHUNK // 2, pair_body, 0)
    out_copy(_NCHUNK - 2, 0).wait()
    out_copy(_NCHUNK - 1, 1).wait()


@jax.jit
def kernel(inputs, table, bias):
    idx = inputs.astype(jnp.int32)
    mesh = plsc.VectorSubcoreMesh(
        core_axis_name="c", subcore_axis_name="s",
        num_cores=_NC, num_subcores=_NS)

    transpose_k = functools.partial(
        pl.kernel,
        out_type=jax.ShapeDtypeStruct((_V // 4, 4 * _D), jnp.float32),
        mesh=mesh,
        scratch_types=[
            pltpu.VMEM((_D, _SB), jnp.float32),
            pltpu.VMEM((_D, _SB), jnp.float32),
            pltpu.VMEM((_SB // 4, 4 * _D), jnp.float32),
            pltpu.VMEM((_SB // 4, 4 * _D), jnp.float32),
            pltpu.SemaphoreType.DMA,
            pltpu.SemaphoreType.DMA,
            pltpu.SemaphoreType.DMA,
            pltpu.SemaphoreType.DMA,
        ],
        compiler_params=pltpu.CompilerParams(
            use_tc_tiling_on_sc=True, needs_layout_passes=False),
    )(_transpose_body)
    tail128 = lax.slice(table, (_NBLK * _SB, 0), (_V, _D)).reshape(16, 128)
    t128 = transpose_k(table.T, tail128)
    table_rm = t128.reshape(_V, _D)

    gather_k = functools.partial(
        pl.kernel,
        out_type=jax.ShapeDtypeStruct((_B, _D), jnp.float32),
        mesh=mesh,
        scratch_types=[
            pltpu.VMEM((_CB, _H), jnp.int32),
            pltpu.VMEM((_CB, _H), jnp.int32),
            pltpu.VMEM((_CB * _H, _D), jnp.float32),
            pltpu.VMEM((_CB * _H, _D), jnp.float32),
            pltpu.VMEM((_CB, _D), jnp.float32),
            pltpu.VMEM((_CB, _D), jnp.float32),
            pltpu.VMEM((_D,), jnp.float32),
            pltpu.SemaphoreType.DMA,
            pltpu.SemaphoreType.DMA,
            pltpu.SemaphoreType.DMA,
            pltpu.SemaphoreType.DMA,
        ],
        compiler_params=pltpu.CompilerParams(use_tc_tiling_on_sc=False),
    )(_bow_body)
    return gather_k(idx, table_rm, bias)


# speedup vs baseline: 1.1076x; 1.1076x over previous
import functools

import jax
import jax.numpy as jnp
from jax import lax
from jax.experimental import pallas as pl
from jax.experimental.pallas import tpu as pltpu
from jax.experimental.pallas import tpu_sc as plsc

_B = 16384
_H = 50
_D = 32
_V = 1000000
_NC = 2
_NS = 16
_NW = _NC * _NS
_BPW = _B // _NW
_CB = 32
_NCHUNK = _BPW // _CB

_RB = 128
_SB = 128
_NBLK = _V // _SB
_TAIL = _V - _NBLK * _SB
_BLK_STATIC = 245
_BLK_LAST = _NBLK - _BLK_STATIC


def _transpose_body(tt_hbm, tail_hbm, out_hbm,
                    sbuf0, sbuf1, obuf0, obuf1,
                    isem0, isem1, osem0, osem1):
    wid = lax.axis_index("s") * _NC + lax.axis_index("c")
    base = jnp.minimum(wid * _BLK_STATIC, _BLK_LAST)
    sbufs = (sbuf0, sbuf1)
    obufs = (obuf0, obuf1)
    isems = (isem0, isem1)
    osems = (osem0, osem1)

    lanes = lax.iota(jnp.int32, 16)
    z = jnp.minimum(wid, 0)

    def in_copy(t, p):
        c0 = pl.multiple_of(t * _SB, _SB)
        return pltpu.make_async_copy(
            tt_hbm.at[:, pl.ds(c0, _SB)], sbufs[p], isems[p])

    def out_copy(t, p):
        s0 = pl.multiple_of(t * (_SB // 4), _SB // 4)
        return pltpu.make_async_copy(
            obufs[p], out_hbm.at[pl.ds(s0, _SB // 4)], osems[p])

    def transpose_block(p):
        sb = sbufs[p]
        ob = obufs[p]

        @plsc.parallel_loop(0, 16, 1, unroll=4)
        def _(d):
            m = jnp.bitwise_and(lanes + d, 15)
            mrow = jnp.right_shift(m, 2)
            mcol = jnp.bitwise_and(m, 3) * _D
            for cg in range(2):
                lc = lanes + cg * 16
                colf = mcol + lc
                srcs = [plsc.load_gather(sb, [lc, m + rg * 16])
                        for rg in range(_SB // 16)]
                for rg in range(_SB // 16):
                    plsc.store_scatter(ob, [mrow + rg * 4, colf], srcs[rg])

    def do_block(t, p, first):
        in_copy(t, p).wait()
        @pl.when(jnp.logical_not(first))
        def _():
            out_copy(t, p).wait()
        transpose_block(p)
        out_copy(t, p).start()
        nxt = t + 2
        @pl.when(nxt < base + _BLK_STATIC)
        def _():
            in_copy(nxt, p).start()

    in_copy(base, 0).start()
    in_copy(base + 1, 1).start()

    def pair_body(j, carry):
        t = base + 2 * j
        do_block(t, 0, j == 0)
        do_block(t + 1, 1, j == 0)
        return carry

    lax.fori_loop(0, _BLK_STATIC // 2, pair_body, 0)
    do_block(base + _BLK_STATIC - 1, 0, False)
    out_copy(base + _BLK_STATIC - 1, 0).wait()
    out_copy(base + _BLK_STATIC - 2, 1).wait()

    @pl.when(wid == _NW - 1)
    def _():
        pltpu.async_copy(tail_hbm, obuf0.at[pl.ds(0, 16)], isem0).wait()
        pltpu.async_copy(obuf0.at[pl.ds(0, 16)],
                         out_hbm.at[pl.ds(_NBLK * (_SB // 4), 16)],
                         isem0).wait()


def _bow_body(idx_hbm, table_hbm, bias_hbm, out_hbm,
              idx0, idx1, rows0, rows1, outv0, outv1, bias_v,
              gsem0, gsem1, osem0, osem1):
    wid = lax.axis_index("s") * _NC + lax.axis_index("c")
    base_row = wid * _BPW
    idxs = (idx0, idx1)
    rows = (rows0, rows1)
    outs = (outv0, outv1)
    gsems = (gsem0, gsem1)
    osems = (osem0, osem1)

    pltpu.sync_copy(bias_hbm, bias_v)
    bias0 = bias_v[pl.ds(0, 16)]
    bias1 = bias_v[pl.ds(16, 16)]

    def row0_of(c):
        return pl.multiple_of(base_row + c * _CB, _CB)

    def fire(c, p):
        pltpu.sync_copy(idx_hbm.at[pl.ds(row0_of(c), _CB)], idxs[p])
        for r in range(_CB):
            pltpu.make_async_copy(
                table_hbm.at[idxs[p].at[r]],
                rows[p].at[pl.ds(r * _H, _H)], gsems[p]).start()

    def drain(p):
        for r in range(_CB):
            pltpu.make_async_copy(
                table_hbm.at[idxs[p].at[r]],
                rows[p].at[pl.ds(r * _H, _H)], gsems[p]).wait()

    def out_copy(c, p):
        return pltpu.make_async_copy(
            outs[p], out_hbm.at[pl.ds(row0_of(c), _CB)], osems[p])

    def accumulate(p):
        rv = rows[p]
        ov = outs[p]

        @plsc.parallel_loop(0, _CB, 1, unroll=2)
        def _(b):
            r0 = b * _H
            a0 = bias0
            a1 = bias1
            b0 = rv[r0, pl.ds(0, 16)]
            b1 = rv[r0, pl.ds(16, 16)]
            for h in range(1, _H, 2):
                a0 = a0 + rv[r0 + h, pl.ds(0, 16)]
                a1 = a1 + rv[r0 + h, pl.ds(16, 16)]
                if h + 1 < _H:
                    b0 = b0 + rv[r0 + h + 1, pl.ds(0, 16)]
                    b1 = b1 + rv[r0 + h + 1, pl.ds(16, 16)]
            ov[b, pl.ds(0, 16)] = a0 + b0
            ov[b, pl.ds(16, 16)] = a1 + b1

    def step(c, p, first):
        drain(p)
        @pl.when(c + 1 < _NCHUNK)
        def _():
            fire(c + 1, 1 - p)
        @pl.when(jnp.logical_not(first))
        def _():
            out_copy(c, p).wait()
        accumulate(p)
        out_copy(c, p).start()

    fire(0, 0)

    def pair_body(j, carry):
        c = 2 * j
        step(c, 0, j == 0)
        step(c + 1, 1, j == 0)
        return carry

    lax.fori_loop(0, _NCHUNK // 2, pair_body, 0)
    out_copy(_NCHUNK - 2, 0).wait()
    out_copy(_NCHUNK - 1, 1).wait()


@jax.jit
def kernel(inputs, table, bias):
    idx = inputs.astype(jnp.int32)
    mesh = plsc.VectorSubcoreMesh(
        core_axis_name="c", subcore_axis_name="s",
        num_cores=_NC, num_subcores=_NS)

    transpose_k = functools.partial(
        pl.kernel,
        out_type=jax.ShapeDtypeStruct((_V // 4, 4 * _D), jnp.float32),
        mesh=mesh,
        scratch_types=[
            pltpu.VMEM((_D, _SB), jnp.float32),
            pltpu.VMEM((_D, _SB), jnp.float32),
            pltpu.VMEM((_SB // 4, 4 * _D), jnp.float32),
            pltpu.VMEM((_SB // 4, 4 * _D), jnp.float32),
            pltpu.SemaphoreType.DMA,
            pltpu.SemaphoreType.DMA,
            pltpu.SemaphoreType.DMA,
            pltpu.SemaphoreType.DMA,
        ],
        compiler_params=pltpu.CompilerParams(
            use_tc_tiling_on_sc=True, needs_layout_passes=False),
    )(_transpose_body)
    tail128 = lax.slice(table, (_NBLK * _SB, 0), (_V, _D)).reshape(16, 128)
    t128 = transpose_k(table.T, tail128)
    table_rm = t128.reshape(_V, _D)

    gather_k = functools.partial(
        pl.kernel,
        out_type=jax.ShapeDtypeStruct((_B, _D), jnp.float32),
        mesh=mesh,
        scratch_types=[
            pltpu.VMEM((_CB, _H), jnp.int32),
            pltpu.VMEM((_CB, _H), jnp.int32),
            pltpu.VMEM((_CB * _H, _D), jnp.float32),
            pltpu.VMEM((_CB * _H, _D), jnp.float32),
            pltpu.VMEM((_CB, _D), jnp.float32),
            pltpu.VMEM((_CB, _D), jnp.float32),
            pltpu.VMEM((_D,), jnp.float32),
            pltpu.SemaphoreType.DMA,
            pltpu.SemaphoreType.DMA,
            pltpu.SemaphoreType.DMA,
            pltpu.SemaphoreType.DMA,
        ],
        compiler_params=pltpu.CompilerParams(use_tc_tiling_on_sc=False),
    )(_bow_body)
    return gather_k(idx, table_rm, bias)


# speedup vs baseline: 1.3999x; 1.2639x over previous
import functools

import jax
import jax.numpy as jnp
from jax import lax
from jax.experimental import pallas as pl
from jax.experimental.pallas import tpu as pltpu
from jax.experimental.pallas import tpu_sc as plsc

_B = 16384
_H = 50
_D = 32
_V = 1000000
_NC = 2
_NS = 16
_NW = _NC * _NS
_BPW = _B // _NW
_CB = 32
_NCHUNK = _BPW // _CB

_RB = 128
_SB = 128
_NBLK = _V // _SB
_TAIL = _V - _NBLK * _SB
_BLK_STATIC = 245
_BLK_LAST = _NBLK - _BLK_STATIC


def _transpose_body(tt_hbm, tail_hbm, out_hbm,
                    sbuf0, sbuf1, sbuf2, sbuf3, obuf0, obuf1, obuf2, obuf3,
                    isem0, isem1, isem2, isem3, osem0, osem1, osem2, osem3):
    wid = lax.axis_index("s") * _NC + lax.axis_index("c")
    base = jnp.minimum(wid * _BLK_STATIC, _BLK_LAST)
    sbufs = (sbuf0, sbuf1, sbuf2, sbuf3)
    obufs = (obuf0, obuf1, obuf2, obuf3)
    isems = (isem0, isem1, isem2, isem3)
    osems = (osem0, osem1, osem2, osem3)

    lanes = lax.iota(jnp.int32, 16)
    z = jnp.minimum(wid, 0)

    def in_copy(t, p):
        c0 = pl.multiple_of(t * _SB, _SB)
        return pltpu.make_async_copy(
            tt_hbm.at[:, pl.ds(c0, _SB)], sbufs[p], isems[p])

    def out_copy(t, p):
        s0 = pl.multiple_of(t * (_SB // 4), _SB // 4)
        return pltpu.make_async_copy(
            obufs[p], out_hbm.at[pl.ds(s0, _SB // 4)], osems[p])

    def transpose_block(p):
        sb = sbufs[p]
        ob = obufs[p]

        @plsc.parallel_loop(0, 16, 1, unroll=4)
        def _(d):
            m = jnp.bitwise_and(lanes + d, 15)
            mrow = jnp.right_shift(m, 2)
            mcol = jnp.bitwise_and(m, 3) * _D
            for cg in range(2):
                lc = lanes + cg * 16
                colf = mcol + lc
                srcs = [plsc.load_gather(sb, [lc, m + rg * 16])
                        for rg in range(_SB // 16)]
                for rg in range(_SB // 16):
                    plsc.store_scatter(ob, [mrow + rg * 4, colf], srcs[rg])

    def do_block(t, p, first):
        in_copy(t, p).wait()
        @pl.when(jnp.logical_not(first))
        def _():
            out_copy(t, p).wait()
        transpose_block(p)
        out_copy(t, p).start()
        nxt = t + 4
        @pl.when(nxt < base + _BLK_STATIC)
        def _():
            in_copy(nxt, p).start()

    for p in range(4):
        in_copy(base + p, p).start()

    def quad_body(j, carry):
        t = base + 4 * j
        for p in range(4):
            do_block(t + p, p, j == 0)
        return carry

    lax.fori_loop(0, _BLK_STATIC // 4, quad_body, 0)
    do_block(base + _BLK_STATIC - 1, 0, False)
    out_copy(base + _BLK_STATIC - 1, 0).wait()
    for p in range(1, 4):
        out_copy(base + _BLK_STATIC - 4 + p, p).wait()

    @pl.when(wid == _NW - 1)
    def _():
        pltpu.async_copy(tail_hbm, obuf0.at[pl.ds(0, 16)], isem0).wait()
        pltpu.async_copy(obuf0.at[pl.ds(0, 16)],
                         out_hbm.at[pl.ds(_NBLK * (_SB // 4), 16)],
                         isem0).wait()


def _bow_body(idx_hbm, table_hbm, bias_hbm, out_hbm,
              idx0, idx1, rows0, rows1, outv0, outv1, bias_v,
              gsem0, gsem1, osem0, osem1):
    wid = lax.axis_index("s") * _NC + lax.axis_index("c")
    base_row = wid * _BPW
    idxs = (idx0, idx1)
    rows = (rows0, rows1)
    outs = (outv0, outv1)
    gsems = (gsem0, gsem1)
    osems = (osem0, osem1)

    pltpu.sync_copy(bias_hbm, bias_v)
    bias0 = bias_v[pl.ds(0, 16)]
    bias1 = bias_v[pl.ds(16, 16)]

    def row0_of(c):
        return pl.multiple_of(base_row + c * _CB, _CB)

    def fire(c, p):
        pltpu.sync_copy(idx_hbm.at[pl.ds(row0_of(c), _CB)], idxs[p])
        for r in range(_CB):
            pltpu.make_async_copy(
                table_hbm.at[idxs[p].at[r]],
                rows[p].at[pl.ds(r * _H, _H)], gsems[p]).start()

    def drain(p):
        for r in range(_CB):
            pltpu.make_async_copy(
                table_hbm.at[idxs[p].at[r]],
                rows[p].at[pl.ds(r * _H, _H)], gsems[p]).wait()

    def out_copy(c, p):
        return pltpu.make_async_copy(
            outs[p], out_hbm.at[pl.ds(row0_of(c), _CB)], osems[p])

    def accumulate(p):
        rv = rows[p]
        ov = outs[p]

        @plsc.parallel_loop(0, _CB, 1, unroll=2)
        def _(b):
            r0 = b * _H
            a0 = bias0
            a1 = bias1
            b0 = rv[r0, pl.ds(0, 16)]
            b1 = rv[r0, pl.ds(16, 16)]
            for h in range(1, _H, 2):
                a0 = a0 + rv[r0 + h, pl.ds(0, 16)]
                a1 = a1 + rv[r0 + h, pl.ds(16, 16)]
                if h + 1 < _H:
                    b0 = b0 + rv[r0 + h + 1, pl.ds(0, 16)]
                    b1 = b1 + rv[r0 + h + 1, pl.ds(16, 16)]
            ov[b, pl.ds(0, 16)] = a0 + b0
            ov[b, pl.ds(16, 16)] = a1 + b1

    def step(c, p, first):
        drain(p)
        @pl.when(c + 1 < _NCHUNK)
        def _():
            fire(c + 1, 1 - p)
        @pl.when(jnp.logical_not(first))
        def _():
            out_copy(c, p).wait()
        accumulate(p)
        out_copy(c, p).start()

    fire(0, 0)

    def pair_body(j, carry):
        c = 2 * j
        step(c, 0, j == 0)
        step(c + 1, 1, j == 0)
        return carry

    lax.fori_loop(0, _NCHUNK // 2, pair_body, 0)
    out_copy(_NCHUNK - 2, 0).wait()
    out_copy(_NCHUNK - 1, 1).wait()


@jax.jit
def kernel(inputs, table, bias):
    idx = inputs.astype(jnp.int32)
    mesh = plsc.VectorSubcoreMesh(
        core_axis_name="c", subcore_axis_name="s",
        num_cores=_NC, num_subcores=_NS)

    transpose_k = functools.partial(
        pl.kernel,
        out_type=jax.ShapeDtypeStruct((_V // 4, 4 * _D), jnp.float32),
        mesh=mesh,
        scratch_types=(
            [pltpu.VMEM((_D, _SB), jnp.float32)] * 4
            + [pltpu.VMEM((_SB // 4, 4 * _D), jnp.float32)] * 4
            + [pltpu.SemaphoreType.DMA] * 8
        ),
        compiler_params=pltpu.CompilerParams(
            use_tc_tiling_on_sc=True, needs_layout_passes=False),
    )(_transpose_body)
    tail128 = lax.slice(table, (_NBLK * _SB, 0), (_V, _D)).reshape(16, 128)
    t128 = transpose_k(table.T, tail128)
    table_rm = t128.reshape(_V, _D)

    gather_k = functools.partial(
        pl.kernel,
        out_type=jax.ShapeDtypeStruct((_B, _D), jnp.float32),
        mesh=mesh,
        scratch_types=[
            pltpu.VMEM((_CB, _H), jnp.int32),
            pltpu.VMEM((_CB, _H), jnp.int32),
            pltpu.VMEM((_CB * _H, _D), jnp.float32),
            pltpu.VMEM((_CB * _H, _D), jnp.float32),
            pltpu.VMEM((_CB, _D), jnp.float32),
            pltpu.VMEM((_CB, _D), jnp.float32),
            pltpu.VMEM((_D,), jnp.float32),
            pltpu.SemaphoreType.DMA,
            pltpu.SemaphoreType.DMA,
            pltpu.SemaphoreType.DMA,
            pltpu.SemaphoreType.DMA,
        ],
        compiler_params=pltpu.CompilerParams(use_tc_tiling_on_sc=False),
    )(_bow_body)
    return gather_k(idx, table_rm, bias)
